# baseline (speedup 1.0000x reference)
import functools
import math

import jax
import jax.numpy as jnp
from jax import lax
from jax.experimental import pallas as pl
from jax.experimental.pallas import tpu as pltpu
from jax.experimental.pallas import tpu_sc as plsc

_VOCAB = 100000
_D = 128
_B = 4096
_L = 50
_NTOK = _B * _L
_NC = 2
_NS = 16
_NW = _NC * _NS
_CHUNK = 128
_ROWS_PER_W = _NTOK // _NW
_NCH = _ROWS_PER_W // _CHUNK
_NBUF = 5
_LAG = 2
_SCALE = math.sqrt(float(_D))


def _emb_kernel(idx_hbm, table_hbm, out_hbm, idx_v, *scratch):
    rows = scratch[:_NBUF]
    gsem = scratch[_NBUF:2 * _NBUF]
    ssem = scratch[2 * _NBUF:3 * _NBUF]

    wid = lax.axis_index("s") * _NC + lax.axis_index("c")
    tok0 = wid * _ROWS_PER_W

    pltpu.sync_copy(idx_hbm.at[pl.ds(tok0, _ROWS_PER_W)], idx_v)

    def start_gather(j, b):
        off = pl.multiple_of(j * _CHUNK, _CHUNK)
        pltpu.async_copy(
            table_hbm.at[idx_v.at[pl.ds(off, _CHUNK)]], rows[b], gsem[b]
        )

    def out_slice(j):
        row0 = pl.multiple_of(tok0 + j * _CHUNK, _CHUNK)
        return out_hbm.at[pl.ds(row0, _CHUNK)]

    for b in range(_NBUF):
        start_gather(b, b)

    def outer(g, carry):
        for b in range(_NBUF):
            j = g * _NBUF + b
            pltpu.make_async_copy(
                table_hbm.at[idx_v.at[pl.ds(0, _CHUNK)]], rows[b], gsem[b]
            ).wait()

            def scale_row(r, c2, _b=b):
                for c in range(_D // 16):
                    rows[_b][r, pl.ds(c * 16, 16)] = (
                        rows[_b][r, pl.ds(c * 16, 16)] * _SCALE
                    )
                return c2

            lax.fori_loop(0, _CHUNK, scale_row, 0, unroll=False)

            pltpu.async_copy(rows[b], out_slice(j), ssem[b])

            jr = j - _LAG
            br = (b - _LAG) % _NBUF
            cond = (jr >= 0) & (jr < _NCH - _NBUF)
            @pl.when(cond)
            def _(jr=jr, br=br):
                pltpu.make_async_copy(
                    rows[br], out_slice(jr), ssem[br]
                ).wait()
                start_gather(jr + _NBUF, br)
        return carry

    lax.fori_loop(0, _NCH // _NBUF, outer, 0, unroll=False)

    for b in range(_NBUF):
        pltpu.make_async_copy(rows[b], out_slice(0), ssem[b]).wait()


@jax.jit
def _run(x2, table):
    mesh = plsc.VectorSubcoreMesh(core_axis_name="c", subcore_axis_name="s")
    f = functools.partial(
        pl.kernel,
        out_type=jax.ShapeDtypeStruct((_NTOK, _D), jnp.float32),
        mesh=mesh,
        scratch_types=[pltpu.VMEM((_ROWS_PER_W,), jnp.int32)]
        + [pltpu.VMEM((_CHUNK, _D), jnp.float32) for _ in range(_NBUF)]
        + [pltpu.SemaphoreType.DMA for _ in range(2 * _NBUF)],
    )(_emb_kernel)
    return f(x2, table)


def kernel(x, table):
    x2 = x.reshape(_NTOK)
    out = _run(x2, table)
    return out.reshape(_B, _L, _D)


# speedup vs baseline: 1.7620x; 1.7620x over previous
import functools
import math

import jax
import jax.numpy as jnp
from jax import lax
from jax.experimental import pallas as pl
from jax.experimental.pallas import tpu as pltpu
from jax.experimental.pallas import tpu_sc as plsc

_VOCAB = 100000
_D = 128
_B = 4096
_L = 50
_NC = 2
_NS = 16
_NW = _NC * _NS
_BPW = _B // _NW
_NB = 4
_NGRP = _BPW // _NB
_NBUF = 4
_LAG = 2
_SCALE = math.sqrt(float(_D))


def _emb_kernel(idx_hbm, table_hbm, out_hbm, idx_v, *scratch):
    rows = scratch[:_NBUF]
    gsem = scratch[_NBUF:2 * _NBUF]
    ssem = scratch[2 * _NBUF:3 * _NBUF]

    wid = lax.axis_index("s") * _NC + lax.axis_index("c")
    row0 = wid * _BPW

    pltpu.sync_copy(idx_hbm.at[pl.ds(row0 * 1, _BPW)], idx_v)

    def start_gathers(g, b):
        for i in range(_NB):
            pltpu.async_copy(
                table_hbm.at[idx_v.at[g * _NB + i]], rows[b].at[i], gsem[b]
            )

    def wait_gathers(b):
        pltpu.make_async_copy(
            table_hbm.at[idx_v.at[0]], rows[b], gsem[b]
        ).wait()

    def out_slice(g):
        return out_hbm.at[pl.ds(row0 + g * _NB, _NB)]

    for b in range(_NBUF):
        start_gathers(b, b)

    def outer(gg, carry):
        for b in range(_NBUF):
            g = gg * _NBUF + b
            wait_gathers(b)

            def scale_row(r, c2, _b=b):
                for i in range(_NB):
                    for c in range(_D // 16):
                        rows[_b][i, r, pl.ds(c * 16, 16)] = (
                            rows[_b][i, r, pl.ds(c * 16, 16)] * _SCALE
                        )
                return c2

            lax.fori_loop(0, _L, scale_row, 0, unroll=False)

            pltpu.async_copy(rows[b], out_slice(g), ssem[b])

            gr = g - _LAG
            br = (b - _LAG) % _NBUF
            cond = (gr >= 0) & (gr < _NGRP - _NBUF)

            @pl.when(cond)
            def _(gr=gr, br=br):
                pltpu.make_async_copy(rows[br], out_slice(gr), ssem[br]).wait()
                start_gathers(gr + _NBUF, br)
        return carry

    lax.fori_loop(0, _NGRP // _NBUF, outer, 0, unroll=False)

    for b in range(_NBUF):
        pltpu.make_async_copy(rows[b], out_slice(0), ssem[b]).wait()


@jax.jit
def _run(x, table):
    mesh = plsc.VectorSubcoreMesh(core_axis_name="c", subcore_axis_name="s")
    f = functools.partial(
        pl.kernel,
        out_type=jax.ShapeDtypeStruct((_B, _L, _D), jnp.float32),
        mesh=mesh,
        scratch_types=[pltpu.VMEM((_BPW, _L), jnp.int32)]
        + [pltpu.VMEM((_NB, _L, _D), jnp.float32) for _ in range(_NBUF)]
        + [pltpu.SemaphoreType.DMA for _ in range(2 * _NBUF)],
    )(_emb_kernel)
    return f(x, table)


def kernel(x, table):
    return _run(x, table)


# speedup vs baseline: 3.1688x; 1.7984x over previous
import functools
import math

import jax
import jax.numpy as jnp
from jax import lax
from jax.experimental import pallas as pl
from jax.experimental.pallas import tpu as pltpu
from jax.experimental.pallas import tpu_sc as plsc

_VOCAB = 100000
_D = 128
_B = 4096
_L = 50
_NC = 2
_NS = 16
_NW = _NC * _NS
_BPW = _B // _NW
_NCH = _L
_NBUF = 5
_LAG = 2
_SCALE = math.sqrt(float(_D))


def _emb_kernel(idx_hbm, table_hbm, out_hbm, idx_v, *scratch):
    rows = scratch[:_NBUF]
    gsem = scratch[_NBUF:2 * _NBUF]
    ssem = scratch[2 * _NBUF:3 * _NBUF]

    wid = lax.axis_index("s") * _NC + lax.axis_index("c")
    b0 = wid * _BPW

    pltpu.sync_copy(idx_hbm.at[pl.ds(0, _L), pl.ds(b0, _BPW)], idx_v)

    def start_gather(l, b):
        pltpu.async_copy(table_hbm.at[idx_v.at[l]], rows[b], gsem[b])

    def out_slice(l):
        return out_hbm.at[l, pl.ds(b0, _BPW)]

    for b in range(_NBUF):
        start_gather(b, b)

    def outer(g, carry):
        for b in range(_NBUF):
            l = g * _NBUF + b
            pltpu.make_async_copy(
                table_hbm.at[idx_v.at[0]], rows[b], gsem[b]
            ).wait()

            def scale_row(r, c2, _b=b):
                for c in range(_D // 16):
                    rows[_b][r, pl.ds(c * 16, 16)] = (
                        rows[_b][r, pl.ds(c * 16, 16)] * _SCALE
                    )
                return c2

            lax.fori_loop(0, _BPW, scale_row, 0, unroll=False)

            pltpu.async_copy(rows[b], out_slice(l), ssem[b])

            lr = l - _LAG
            br = (b - _LAG) % _NBUF
            cond = (lr >= 0) & (lr < _NCH - _NBUF)

            @pl.when(cond)
            def _(lr=lr, br=br):
                pltpu.make_async_copy(rows[br], out_slice(lr), ssem[br]).wait()
                start_gather(lr + _NBUF, br)
        return carry

    lax.fori_loop(0, _NCH // _NBUF, outer, 0, unroll=False)

    for b in range(_NBUF):
        pltpu.make_async_copy(rows[b], out_slice(0), ssem[b]).wait()


@jax.jit
def _run(x_t, table):
    mesh = plsc.VectorSubcoreMesh(core_axis_name="c", subcore_axis_name="s")
    f = functools.partial(
        pl.kernel,
        out_type=jax.ShapeDtypeStruct((_L, _B, _D), jnp.float32),
        mesh=mesh,
        scratch_types=[pltpu.VMEM((_L, _BPW), jnp.int32)]
        + [pltpu.VMEM((_BPW, _D), jnp.float32) for _ in range(_NBUF)]
        + [pltpu.SemaphoreType.DMA for _ in range(2 * _NBUF)],
    )(_emb_kernel)
    return f(x_t, table)


def kernel(x, table):
    out_lbd = _run(x.T, table)
    return jnp.transpose(out_lbd, (1, 0, 2))


# speedup vs baseline: 3.1869x; 1.0057x over previous
import functools
import math

import jax
import jax.numpy as jnp
from jax import lax
from jax.experimental import pallas as pl
from jax.experimental.pallas import tpu as pltpu
from jax.experimental.pallas import tpu_sc as plsc

_VOCAB = 100000
_D = 128
_B = 4096
_L = 50
_NC = 2
_NS = 16
_NW = _NC * _NS
_BPW = _B // _NW
_NCH = _L
_NBUF = 5
_LAG = 1
_SCALE = math.sqrt(float(_D))


def _emb_kernel(idx_hbm, table_hbm, out_hbm, idx_v, *scratch):
    rows = scratch[:_NBUF]
    gsem = scratch[_NBUF:2 * _NBUF]
    ssem = scratch[2 * _NBUF:3 * _NBUF]

    wid = lax.axis_index("s") * _NC + lax.axis_index("c")
    b0 = wid * _BPW

    pltpu.sync_copy(idx_hbm.at[pl.ds(0, _L), pl.ds(b0, _BPW)], idx_v)

    def start_gather(l, b):
        pltpu.async_copy(table_hbm.at[idx_v.at[l]], rows[b], gsem[b])

    def out_slice(l):
        return out_hbm.at[l, pl.ds(b0, _BPW)]

    for b in range(_NBUF):
        start_gather(b, b)

    def outer(g, carry):
        for b in range(_NBUF):
            l = g * _NBUF + b
            pltpu.make_async_copy(
                table_hbm.at[idx_v.at[0]], rows[b], gsem[b]
            ).wait()

            def scale_row(r, c2, _b=b):
                for c in range(_D // 16):
                    rows[_b][r, pl.ds(c * 16, 16)] = (
                        rows[_b][r, pl.ds(c * 16, 16)] * _SCALE
                    )
                return c2

            lax.fori_loop(0, _BPW, scale_row, 0, unroll=False)

            pltpu.async_copy(rows[b], out_slice(l), ssem[b])

            lr = l - _LAG
            br = (b - _LAG) % _NBUF
            cond = (lr >= 0) & (lr < _NCH - _NBUF)

            @pl.when(cond)
            def _(lr=lr, br=br):
                pltpu.make_async_copy(rows[br], out_slice(lr), ssem[br]).wait()
                start_gather(lr + _NBUF, br)
        return carry

    lax.fori_loop(0, _NCH // _NBUF, outer, 0, unroll=False)

    for b in range(_NBUF):
        pltpu.make_async_copy(rows[b], out_slice(0), ssem[b]).wait()


@jax.jit
def _run(x_t, table):
    mesh = plsc.VectorSubcoreMesh(core_axis_name="c", subcore_axis_name="s")
    f = functools.partial(
        pl.kernel,
        out_type=jax.ShapeDtypeStruct((_L, _B, _D), jnp.float32),
        mesh=mesh,
        scratch_types=[pltpu.VMEM((_L, _BPW), jnp.int32)]
        + [pltpu.VMEM((_BPW, _D), jnp.float32) for _ in range(_NBUF)]
        + [pltpu.SemaphoreType.DMA for _ in range(2 * _NBUF)],
    )(_emb_kernel)
    return f(x_t, table)


def kernel(x, table):
    out_lbd = _run(x.T, table)
    return jnp.transpose(out_lbd, (1, 0, 2))
